# baseline (speedup 1.0000x reference)
import functools

import jax
import jax.numpy as jnp
from jax import lax
from jax.experimental import pallas as pl
from jax.experimental.pallas import tpu as pltpu
from jax.experimental.pallas import tpu_sc as plsc

_GRP = 4
_BLK = 16 * _GRP


def _round_bf16(v):
    u = lax.bitcast_convert_type(v, jnp.uint32)
    lsb = (u >> jnp.uint32(16)) & jnp.uint32(1)
    u = (u + jnp.uint32(32767) + lsb) & jnp.uint32(0xFFFF0000)
    return lax.bitcast_convert_type(u, jnp.float32)


def _make_sc_call(b, s, d, t):
    n = b * s
    info = plsc.get_sparse_core_info()
    nc, ns = info.num_cores, info.num_subcores
    nw = nc * ns
    tpw = n // nw
    spw = s // (nw // b)
    nblk = tpw // _BLK
    mesh = plsc.VectorSubcoreMesh(core_axis_name="c", subcore_axis_name="s")

    @functools.partial(
        pl.kernel, mesh=mesh,
        out_type=[jax.ShapeDtypeStruct((b, s), jnp.int32),
                  jax.ShapeDtypeStruct((b, s), jnp.int32)],
        scratch_types=[
            pltpu.VMEM((d, tpw), jnp.float32),
            pltpu.VMEM((tpw,), jnp.int32),
            pltpu.VMEM((d, 8 * 16), jnp.float32),
            pltpu.VMEM((16,), jnp.int32),
            pltpu.VMEM((tpw,), jnp.int32),
            pltpu.VMEM((tpw,), jnp.int32),
        ],
    )
    def sc_router(xt_hbm, pos_hbm, w_hbm, half_hbm, sel_hbm, tgt_hbm,
                  xv, posv, wv, halfv, selv, tgtv):
        wid = lax.axis_index("s") * nc + lax.axis_index("c")
        bi = wid // (nw // b)
        s0 = (wid % (nw // b)) * spw
        pltpu.sync_copy(xt_hbm.at[bi, :, pl.ds(s0, tpw)], xv)
        pltpu.sync_copy(pos_hbm.at[bi, pl.ds(s0, tpw)], posv)
        pltpu.sync_copy(w_hbm, wv)
        pltpu.sync_copy(half_hbm, halfv)
        halfvec = halfv[...]

        zero = jnp.zeros((16,), jnp.float32)

        def blk_body(blk, carry):
            base = blk * _BLK

            def c_body(c, accs):
                xcs = [
                    _round_bf16(xv[c, pl.ds(base + g * 16, 16)])
                    for g in range(_GRP)
                ]
                out = []
                for ti in range(t):
                    wtc = wv[c, pl.ds(ti * 16, 16)]
                    for g in range(_GRP):
                        out.append(accs[ti * _GRP + g] + xcs[g] * wtc)
                return out

            accs = lax.fori_loop(0, d, c_body, [zero] * (t * _GRP),
                                 unroll=4)

            for g in range(_GRP):
                best = accs[g]
                arg = jnp.zeros((16,), jnp.int32)
                for ti in range(1, t):
                    a = accs[ti * _GRP + g]
                    m = a > best
                    best = jnp.where(m, a, best)
                    arg = jnp.where(m, jnp.int32(ti), arg)
                off = base + g * 16
                selv[pl.ds(off, 16)] = arg
                pos = posv[pl.ds(off, 16)]
                x0 = xv[0, pl.ds(off, 16)]
                x1 = xv[1, pl.ds(off, 16)]
                tgtv[pl.ds(off, 16)] = (
                    jnp.where(pos >= halfvec, 4, 0)
                    + jnp.where(x0 > 0, 2, 0)
                    + jnp.where(x1 > 0, 1, 0)).astype(jnp.int32)
            return carry

        lax.fori_loop(0, nblk, blk_body, 0)
        pltpu.sync_copy(selv, sel_hbm.at[bi, pl.ds(s0, tpw)])
        pltpu.sync_copy(tgtv, tgt_hbm.at[bi, pl.ds(s0, tpw)])

    return sc_router


def kernel(x, positions, seq_len, content_sigs):
    b, s, d = x.shape
    t = content_sigs.shape[0]
    w = jnp.tanh(content_sigs)
    wu = lax.bitcast_convert_type(w, jnp.uint32)
    wu = (wu + jnp.uint32(32767) + ((wu >> jnp.uint32(16)) & jnp.uint32(1)))         & jnp.uint32(0xFFFF0000)
    wb = lax.bitcast_convert_type(wu, jnp.float32)
    wsp = jnp.broadcast_to(wb.T[:, :, None], (d, t, 16)).reshape(d, t * 16)
    half = ((jnp.asarray(seq_len) + 1) // 2).astype(jnp.int32)
    halfv = jnp.broadcast_to(half, (16,))
    pos = positions.astype(jnp.int32)
    xt = jnp.transpose(x, (0, 2, 1))
    sel, tgt = _make_sc_call(b, s, d, t)(xt, pos, wsp, halfv)
    return sel, tgt


# speedup vs baseline: 1.9354x; 1.9354x over previous
import functools

import jax
import jax.numpy as jnp
from jax import lax
from jax.experimental import pallas as pl
from jax.experimental.pallas import tpu as pltpu
from jax.experimental.pallas import tpu_sc as plsc

_SB = 512
_GRP = 4
_BLK = 16 * _GRP
_SC_FRAC = 1024


def _tc_body(half_ref, w_ref, x_hbm, pos_hbm, sel_hbm, tgt_hbm,
             xv, posv, selv, tgtv, sem_x, sem_p, sem_o):
    i = pl.program_id(0)
    nsteps = pl.num_programs(0)
    slot = jax.lax.rem(i, 2)
    nxt = jax.lax.rem(i + 1, 2)

    def start_in(j, buf):
        pltpu.make_async_copy(
            x_hbm.at[:, :, pl.ds(j * _SB, _SB)], xv.at[buf], sem_x.at[buf]
        ).start()
        pltpu.make_async_copy(
            pos_hbm.at[:, pl.ds(j * _SB, _SB)], posv.at[buf], sem_p.at[buf]
        ).start()

    @pl.when(i == 0)
    def _():
        start_in(0, 0)

    @pl.when(i + 1 < nsteps)
    def _():
        start_in(i + 1, nxt)

    pltpu.make_async_copy(
        x_hbm.at[:, :, pl.ds(i * _SB, _SB)], xv.at[slot], sem_x.at[slot]
    ).wait()
    pltpu.make_async_copy(
        pos_hbm.at[:, pl.ds(i * _SB, _SB)], posv.at[slot], sem_p.at[slot]
    ).wait()

    @pl.when(i >= 2)
    def _():
        pltpu.make_async_copy(
            selv.at[slot], sel_hbm.at[:, pl.ds((i - 2) * _SB, _SB)],
            sem_o.at[slot, 0]).wait()
        pltpu.make_async_copy(
            tgtv.at[slot], tgt_hbm.at[:, pl.ds((i - 2) * _SB, _SB)],
            sem_o.at[slot, 1]).wait()

    w = w_ref[...]
    half = half_ref[0]
    nb = xv.shape[1]
    for b in range(nb):
        xb = xv[slot, b]
        st = lax.dot_general(w, xb, (((1,), (0,)), ((), ())),
                             preferred_element_type=jnp.float32)
        best = st[0:1, :]
        arg = jnp.zeros_like(best, dtype=jnp.int32)
        for t in range(1, 8):
            row = st[t:t + 1, :]
            m = row > best
            best = jnp.where(m, row, best)
            arg = jnp.where(m, t, arg)
        selv[slot, b] = arg.reshape(-1)
        pos = posv[slot, b]
        x0 = xv[slot, b, 0]
        x1 = xv[slot, b, 1]
        tgtv[slot, b] = (jnp.where(pos >= half, 4, 0) + jnp.where(x0 > 0, 2, 0)
                         + jnp.where(x1 > 0, 1, 0)).astype(jnp.int32)

    pltpu.make_async_copy(
        selv.at[slot], sel_hbm.at[:, pl.ds(i * _SB, _SB)],
        sem_o.at[slot, 0]).start()
    pltpu.make_async_copy(
        tgtv.at[slot], tgt_hbm.at[:, pl.ds(i * _SB, _SB)],
        sem_o.at[slot, 1]).start()

    @pl.when(i == nsteps - 1)
    def _():
        other = jax.lax.rem(i + 1, 2)

        @pl.when(nsteps >= 2)
        def _():
            pltpu.make_async_copy(
                selv.at[other], sel_hbm.at[:, pl.ds((i - 1) * _SB, _SB)],
                sem_o.at[other, 0]).wait()
            pltpu.make_async_copy(
                tgtv.at[other], tgt_hbm.at[:, pl.ds((i - 1) * _SB, _SB)],
                sem_o.at[other, 1]).wait()

        pltpu.make_async_copy(
            selv.at[slot], sel_hbm.at[:, pl.ds(i * _SB, _SB)],
            sem_o.at[slot, 0]).wait()
        pltpu.make_async_copy(
            tgtv.at[slot], tgt_hbm.at[:, pl.ds(i * _SB, _SB)],
            sem_o.at[slot, 1]).wait()


def _make_tc_call(b, s_tc, d):
    grid = (s_tc // _SB,)
    return pl.pallas_call(
        _tc_body,
        grid=grid,
        in_specs=[
            pl.BlockSpec(memory_space=pltpu.SMEM),
            pl.BlockSpec((8, d), lambda i: (0, 0)),
            pl.BlockSpec(memory_space=pltpu.MemorySpace.HBM),
            pl.BlockSpec(memory_space=pltpu.MemorySpace.HBM),
        ],
        out_specs=[
            pl.BlockSpec(memory_space=pltpu.MemorySpace.HBM),
            pl.BlockSpec(memory_space=pltpu.MemorySpace.HBM),
        ],
        out_shape=[
            jax.ShapeDtypeStruct((b, s_tc), jnp.int32),
            jax.ShapeDtypeStruct((b, s_tc), jnp.int32),
        ],
        scratch_shapes=[
            pltpu.VMEM((2, b, d, _SB), jnp.float32),
            pltpu.VMEM((2, b, _SB), jnp.int32),
            pltpu.VMEM((2, b, _SB), jnp.int32),
            pltpu.VMEM((2, b, _SB), jnp.int32),
            pltpu.SemaphoreType.DMA((2,)),
            pltpu.SemaphoreType.DMA((2,)),
            pltpu.SemaphoreType.DMA((2, 2)),
        ],
        compiler_params=pltpu.CompilerParams(
            dimension_semantics=("arbitrary",)),
    )


def _round_bf16(v):
    u = lax.bitcast_convert_type(v, jnp.uint32)
    lsb = (u >> jnp.uint32(16)) & jnp.uint32(1)
    u = (u + jnp.uint32(32767) + lsb) & jnp.uint32(0xFFFF0000)
    return lax.bitcast_convert_type(u, jnp.float32)


def _make_sc_call(b, s, d, t, s0_base, s_sc):
    info = plsc.get_sparse_core_info()
    nc, ns = info.num_cores, info.num_subcores
    nw = nc * ns
    npb = nw // b
    spw = s_sc // npb
    nblk = spw // _BLK
    mesh = plsc.VectorSubcoreMesh(core_axis_name="c", subcore_axis_name="s")

    @functools.partial(
        pl.kernel, mesh=mesh,
        out_type=[jax.ShapeDtypeStruct((b, s_sc), jnp.int32),
                  jax.ShapeDtypeStruct((b, s_sc), jnp.int32)],
        scratch_types=[
            pltpu.VMEM((d, spw), jnp.float32),
            pltpu.VMEM((spw,), jnp.int32),
            pltpu.VMEM((d, 8 * 16), jnp.float32),
            pltpu.VMEM((16,), jnp.int32),
            pltpu.VMEM((spw,), jnp.int32),
            pltpu.VMEM((spw,), jnp.int32),
        ],
    )
    def sc_router(xt_hbm, pos_hbm, w_hbm, half_hbm, sel_hbm, tgt_hbm,
                  xv, posv, wv, halfv, selv, tgtv):
        wid = lax.axis_index("s") * nc + lax.axis_index("c")
        bi = wid // npb
        loc = (wid % npb) * spw
        s0 = s0_base + loc
        pltpu.sync_copy(xt_hbm.at[bi, :, pl.ds(s0, spw)], xv)
        pltpu.sync_copy(pos_hbm.at[bi, pl.ds(s0, spw)], posv)
        pltpu.sync_copy(w_hbm, wv)
        pltpu.sync_copy(half_hbm, halfv)
        halfvec = halfv[...]

        zero = jnp.zeros((16,), jnp.float32)

        def blk_body(blk, carry):
            base = blk * _BLK

            def c_body(c, accs):
                xcs = [
                    _round_bf16(xv[c, pl.ds(base + g * 16, 16)])
                    for g in range(_GRP)
                ]
                out = []
                for ti in range(t):
                    wtc = wv[c, pl.ds(ti * 16, 16)]
                    for g in range(_GRP):
                        out.append(accs[ti * _GRP + g] + xcs[g] * wtc)
                return out

            accs = lax.fori_loop(0, d, c_body, [zero] * (t * _GRP))

            for g in range(_GRP):
                best = accs[g]
                arg = jnp.zeros((16,), jnp.int32)
                for ti in range(1, t):
                    a = accs[ti * _GRP + g]
                    m = a > best
                    best = jnp.where(m, a, best)
                    arg = jnp.where(m, jnp.int32(ti), arg)
                off = base + g * 16
                selv[pl.ds(off, 16)] = arg
                pos = posv[pl.ds(off, 16)]
                x0 = xv[0, pl.ds(off, 16)]
                x1 = xv[1, pl.ds(off, 16)]
                tgtv[pl.ds(off, 16)] = (
                    jnp.where(pos >= halfvec, 4, 0)
                    + jnp.where(x0 > 0, 2, 0)
                    + jnp.where(x1 > 0, 1, 0)).astype(jnp.int32)
            return carry

        lax.fori_loop(0, nblk, blk_body, 0)
        pltpu.sync_copy(selv, sel_hbm.at[bi, pl.ds(loc, spw)])
        pltpu.sync_copy(tgtv, tgt_hbm.at[bi, pl.ds(loc, spw)])

    return sc_router


def kernel(x, positions, seq_len, content_sigs):
    b, s, d = x.shape
    t = content_sigs.shape[0]
    w = jnp.tanh(content_sigs)
    wu = lax.bitcast_convert_type(w, jnp.uint32)
    wu = (wu + jnp.uint32(32767) + ((wu >> jnp.uint32(16)) & jnp.uint32(1))) \
        & jnp.uint32(0xFFFF0000)
    wb = lax.bitcast_convert_type(wu, jnp.float32)
    wsp = jnp.broadcast_to(wb.T[:, :, None], (d, t, 16)).reshape(d, t * 16)
    half = ((jnp.asarray(seq_len) + 1) // 2).astype(jnp.int32)
    halfv = jnp.broadcast_to(half, (16,))
    half1 = half.reshape(1)
    pos = positions.astype(jnp.int32)
    xt = jnp.transpose(x, (0, 2, 1))
    xt = pltpu.with_memory_space_constraint(xt, pltpu.MemorySpace.HBM)
    pos = pltpu.with_memory_space_constraint(pos, pltpu.MemorySpace.HBM)

    s_sc = _SC_FRAC
    s_tc = s - s_sc
    sel_b, tgt_b = _make_sc_call(b, s, d, t, s_tc, s_sc)(xt, pos, wsp, halfv)
    sel_a, tgt_a = _make_tc_call(b, s_tc, d)(half1, wb, xt, pos)
    sel = jnp.concatenate([sel_a, sel_b], axis=1)
    tgt = jnp.concatenate([tgt_a, tgt_b], axis=1)
    return sel, tgt


# speedup vs baseline: 5.8107x; 3.0024x over previous
import jax
import jax.numpy as jnp
from jax import lax
from jax.experimental import pallas as pl
from jax.experimental.pallas import tpu as pltpu

_SB = 1024


def _tc_body(half_ref, w_ref, x_hbm, pos_hbm, sel_hbm, tgt_hbm,
             xv, posv, selv, tgtv, sem_x, sem_p, sem_o):
    i = pl.program_id(0)
    nsteps = pl.num_programs(0)
    slot = jax.lax.rem(i, 2)
    nxt = jax.lax.rem(i + 1, 2)

    def start_in(j, buf):
        pltpu.make_async_copy(
            x_hbm.at[:, :, pl.ds(j * _SB, _SB)], xv.at[buf], sem_x.at[buf]
        ).start()
        pltpu.make_async_copy(
            pos_hbm.at[:, pl.ds(j * _SB, _SB)], posv.at[buf], sem_p.at[buf]
        ).start()

    @pl.when(i == 0)
    def _():
        start_in(0, 0)

    @pl.when(i + 1 < nsteps)
    def _():
        start_in(i + 1, nxt)

    pltpu.make_async_copy(
        x_hbm.at[:, :, pl.ds(i * _SB, _SB)], xv.at[slot], sem_x.at[slot]
    ).wait()
    pltpu.make_async_copy(
        pos_hbm.at[:, pl.ds(i * _SB, _SB)], posv.at[slot], sem_p.at[slot]
    ).wait()

    @pl.when(i >= 2)
    def _():
        pltpu.make_async_copy(
            selv.at[slot], sel_hbm.at[:, pl.ds((i - 2) * _SB, _SB)],
            sem_o.at[slot, 0]).wait()
        pltpu.make_async_copy(
            tgtv.at[slot], tgt_hbm.at[:, pl.ds((i - 2) * _SB, _SB)],
            sem_o.at[slot, 1]).wait()

    w = w_ref[...]
    half = half_ref[0]
    nb = xv.shape[1]
    for b in range(nb):
        xb = xv[slot, b]
        st = lax.dot_general(w, xb, (((1,), (0,)), ((), ())),
                             preferred_element_type=jnp.float32)
        best = st[0:1, :]
        arg = jnp.zeros_like(best, dtype=jnp.int32)
        for t in range(1, 8):
            row = st[t:t + 1, :]
            m = row > best
            best = jnp.where(m, row, best)
            arg = jnp.where(m, t, arg)
        selv[slot, b] = arg.reshape(-1)
        pos = posv[slot, b]
        x0 = xv[slot, b, 0]
        x1 = xv[slot, b, 1]
        tgtv[slot, b] = (jnp.where(pos >= half, 4, 0) + jnp.where(x0 > 0, 2, 0)
                         + jnp.where(x1 > 0, 1, 0)).astype(jnp.int32)

    pltpu.make_async_copy(
        selv.at[slot], sel_hbm.at[:, pl.ds(i * _SB, _SB)],
        sem_o.at[slot, 0]).start()
    pltpu.make_async_copy(
        tgtv.at[slot], tgt_hbm.at[:, pl.ds(i * _SB, _SB)],
        sem_o.at[slot, 1]).start()

    @pl.when(i == nsteps - 1)
    def _():
        other = jax.lax.rem(i + 1, 2)

        @pl.when(nsteps >= 2)
        def _():
            pltpu.make_async_copy(
                selv.at[other], sel_hbm.at[:, pl.ds((i - 1) * _SB, _SB)],
                sem_o.at[other, 0]).wait()
            pltpu.make_async_copy(
                tgtv.at[other], tgt_hbm.at[:, pl.ds((i - 1) * _SB, _SB)],
                sem_o.at[other, 1]).wait()

        pltpu.make_async_copy(
            selv.at[slot], sel_hbm.at[:, pl.ds(i * _SB, _SB)],
            sem_o.at[slot, 0]).wait()
        pltpu.make_async_copy(
            tgtv.at[slot], tgt_hbm.at[:, pl.ds(i * _SB, _SB)],
            sem_o.at[slot, 1]).wait()


def kernel(x, positions, seq_len, content_sigs):
    b, s, d = x.shape
    w = jnp.tanh(content_sigs)
    half = ((jnp.asarray(seq_len) + 1) // 2).astype(jnp.int32).reshape(1)
    pos = positions.astype(jnp.int32)
    xt = jnp.transpose(x, (0, 2, 1))

    grid = (s // _SB,)
    sel, tgt = pl.pallas_call(
        _tc_body,
        grid=grid,
        in_specs=[
            pl.BlockSpec(memory_space=pltpu.SMEM),
            pl.BlockSpec((8, d), lambda i: (0, 0)),
            pl.BlockSpec(memory_space=pltpu.MemorySpace.HBM),
            pl.BlockSpec(memory_space=pltpu.MemorySpace.HBM),
        ],
        out_specs=[
            pl.BlockSpec(memory_space=pltpu.MemorySpace.HBM),
            pl.BlockSpec(memory_space=pltpu.MemorySpace.HBM),
        ],
        out_shape=[
            jax.ShapeDtypeStruct((b, s), jnp.int32),
            jax.ShapeDtypeStruct((b, s), jnp.int32),
        ],
        scratch_shapes=[
            pltpu.VMEM((2, b, d, _SB), jnp.float32),
            pltpu.VMEM((2, b, _SB), jnp.int32),
            pltpu.VMEM((2, b, _SB), jnp.int32),
            pltpu.VMEM((2, b, _SB), jnp.int32),
            pltpu.SemaphoreType.DMA((2,)),
            pltpu.SemaphoreType.DMA((2,)),
            pltpu.SemaphoreType.DMA((2, 2)),
        ],
        compiler_params=pltpu.CompilerParams(
            dimension_semantics=("arbitrary",)),
    )(half, w,
      pltpu.with_memory_space_constraint(xt, pltpu.MemorySpace.HBM),
      pltpu.with_memory_space_constraint(pos, pltpu.MemorySpace.HBM))
    return sel, tgt
